# baseline (speedup 1.0000x reference)
import functools

import jax
import jax.numpy as jnp
from jax import lax
from jax.experimental import pallas as pl
from jax.experimental.pallas import tpu as pltpu
from jax.experimental.pallas import tpu_sc as plsc

F32 = jnp.float32
I32 = jnp.int32

N = 50000
E = 800000
NODE_IN = 163
H = 64
HH = H // 2
G = 512

NPAD = 50176
EC = 400
E_PER_TILE = E // 16
E_CHUNKS = E_PER_TILE // EC
W_PER_DEG = E // 32
DEG_EC = 1000
DEG_CHUNKS = W_PER_DEG // DEG_EC
NSLICE = NPAD // 16
PC = 784
P_CHUNKS = NSLICE // PC
GP = 520

_mesh = plsc.VectorSubcoreMesh(core_axis_name="c", subcore_axis_name="s")


@functools.partial(
    pl.kernel,
    out_type=jax.ShapeDtypeStruct((NPAD,), F32),
    mesh=_mesh,
    compiler_params=pltpu.CompilerParams(use_tc_tiling_on_sc=False),
    scratch_types=[
        pltpu.VMEM((DEG_EC,), I32),
        pltpu.VMEM((DEG_EC,), F32),
        pltpu.VMEM_SHARED((NPAD,), F32),
    ],
)
def _deg_kernel(ei_hbm, zeros1_hbm, ones_hbm, deg_hbm,
                didx_v, ones_v, acc_sh):
    cid = lax.axis_index("c")
    sid = lax.axis_index("s")

    @pl.when(cid == 0)
    def _():
        pltpu.sync_copy(zeros1_hbm, acc_sh.at[pl.ds(sid * NSLICE, NSLICE)])
        pltpu.sync_copy(ones_hbm, ones_v)

    plsc.subcore_barrier()

    @pl.when(cid == 0)
    def _():
        def body(i, carry):
            base = sid * E_PER_TILE + i * DEG_EC
            pltpu.sync_copy(ei_hbm.at[1, pl.ds(base, DEG_EC)], didx_v)
            pltpu.sync_copy(ones_v, acc_sh.at[didx_v], add=True)
            return carry

        lax.fori_loop(0, E_PER_TILE // DEG_EC, body, 0)

    plsc.subcore_barrier()

    @pl.when(cid == 0)
    def _():
        sl = pl.ds(sid * NSLICE, NSLICE)
        pltpu.sync_copy(acc_sh.at[sl], deg_hbm.at[sl])


@functools.partial(
    pl.kernel,
    out_type=(jax.ShapeDtypeStruct((NPAD, HH), F32),
              jax.ShapeDtypeStruct((NPAD, HH), F32)),
    mesh=_mesh,
    compiler_params=pltpu.CompilerParams(use_tc_tiling_on_sc=False),
    scratch_types=[
        pltpu.VMEM((EC,), I32),
        pltpu.VMEM((EC,), I32),
        pltpu.VMEM((EC,), I32),
        pltpu.VMEM((EC,), I32),
        pltpu.VMEM((EC,), I32),
        pltpu.VMEM((EC,), I32),
        pltpu.VMEM((EC,), I32),
        pltpu.VMEM((EC,), I32),
        pltpu.VMEM((EC, HH), F32),
        pltpu.VMEM((EC, HH), F32),
        pltpu.VMEM_SHARED((NPAD, HH), F32),
        pltpu.SemaphoreType.DMA,
        pltpu.SemaphoreType.DMA,
        pltpu.SemaphoreType.DMA,
        pltpu.SemaphoreType.DMA,
        pltpu.SemaphoreType.DMA,
        pltpu.SemaphoreType.DMA,
        pltpu.SemaphoreType.DMA,
        pltpu.SemaphoreType.DMA,
    ],
)
def _edge_kernel(ei_hbm, xsA_hbm, xsB_hbm, zeros2_hbm,
                 outA_hbm, outB_hbm,
                 sidx0, didx0, sidx1, didx1, sidx2, didx2, sidx3, didx3,
                 rows0, rows1, acc_sh,
                 isem0, isem1, isem2, isem3, gsem0, gsem1, ssem0, ssem1):
    cid = lax.axis_index("c")
    sid = lax.axis_index("s")
    pltpu.sync_copy(zeros2_hbm, acc_sh.at[pl.ds(sid * NSLICE, NSLICE)])
    plsc.subcore_barrier()

    sidx = (sidx0, sidx1, sidx2, sidx3)
    didx = (didx0, didx1, didx2, didx3)
    rows = (rows0, rows1)
    isem = (isem0, isem1, isem2, isem3)
    gsem = (gsem0, gsem1)
    ssem = (ssem0, ssem1)

    def run(xs_hbm):
        def I(c, b):
            base = sid * E_PER_TILE + c * EC
            pltpu.async_copy(ei_hbm.at[0, pl.ds(base, EC)], sidx[b], isem[b])
            pltpu.async_copy(ei_hbm.at[1, pl.ds(base, EC)], didx[b], isem[b])

        def Iw(c, b):
            @pl.when(c < E_CHUNKS)
            def _():
                I(c, b)

        def Gstart(c, b, r):
            base = sid * E_PER_TILE + c * EC
            pltpu.make_async_copy(ei_hbm.at[0, pl.ds(base, EC)], sidx[b],
                                  isem[b]).wait()
            pltpu.make_async_copy(ei_hbm.at[1, pl.ds(base, EC)], didx[b],
                                  isem[b]).wait()
            pltpu.async_copy(xs_hbm.at[sidx[b]], rows[r], gsem[r])

        def Sstart(c, b, r):
            pltpu.make_async_copy(xs_hbm.at[sidx[b]], rows[r],
                                  gsem[r]).wait()
            pltpu.async_copy(rows[r], acc_sh.at[didx[b]], ssem[r], add=True)

        def Swait(c, b, r):
            pltpu.make_async_copy(rows[r], acc_sh.at[didx[b]],
                                  ssem[r]).wait()

        I(0, 0)
        I(1, 1)
        I(2, 2)
        Gstart(0, 0, 0)

        def body(p, carry):
            q = 4 * p
            I(q + 3, 3)
            Gstart(q + 1, 1, 1)
            Sstart(q, 0, 0)
            Sstart(q + 1, 1, 1)
            Swait(q, 0, 0)
            I(q + 4, 0)
            Gstart(q + 2, 2, 0)
            Swait(q + 1, 1, 1)
            Iw(q + 5, 1)
            Gstart(q + 3, 3, 1)
            Sstart(q + 2, 2, 0)
            Sstart(q + 3, 3, 1)
            Swait(q + 2, 2, 0)
            Iw(q + 6, 2)
            Gstart(q + 4, 0, 0)
            Swait(q + 3, 3, 1)
            return carry

        lax.fori_loop(0, (E_CHUNKS - 1) // 4, body, 0)
        Sstart(E_CHUNKS - 1, 0, 0)
        Swait(E_CHUNKS - 1, 0, 0)

    @pl.when(cid == 0)
    def _():
        run(xsA_hbm)

    @pl.when(cid == 1)
    def _():
        run(xsB_hbm)

    plsc.subcore_barrier()
    sl = pl.ds(sid * NSLICE, NSLICE)

    @pl.when(cid == 0)
    def _():
        pltpu.sync_copy(acc_sh.at[sl], outA_hbm.at[sl])

    @pl.when(cid == 1)
    def _():
        pltpu.sync_copy(acc_sh.at[sl], outB_hbm.at[sl])


@functools.partial(
    pl.kernel,
    out_type=(jax.ShapeDtypeStruct((G, HH), F32),
              jax.ShapeDtypeStruct((G, HH), F32),
              jax.ShapeDtypeStruct((G, HH), F32)),
    mesh=_mesh,
    compiler_params=pltpu.CompilerParams(use_tc_tiling_on_sc=False),
    scratch_types=[
        pltpu.VMEM((PC,), I32),
        pltpu.VMEM((PC, HH), F32),
        pltpu.VMEM((PC, HH), F32),
        pltpu.VMEM_SHARED((GP, HH), F32),
        pltpu.VMEM_SHARED((GP, HH), F32),
    ],
)
def _pool_kernel(hA_hbm, hB_hbm, batch_hbm, zeros2_hbm, ones_hbm,
                 sumsA_hbm, sumsB_hbm, cnt_hbm,
                 bidx_v, rows_v, ones_v, accP_sh, accC_sh):
    cid = lax.axis_index("c")
    sid = lax.axis_index("s")

    @pl.when(sid == 0)
    def _():
        pltpu.sync_copy(zeros2_hbm.at[pl.ds(0, GP)], accP_sh)
        pltpu.sync_copy(zeros2_hbm.at[pl.ds(0, GP)], accC_sh)

    pltpu.sync_copy(ones_hbm, ones_v)
    plsc.subcore_barrier()

    def run(h_hbm, do_cnt):
        def body(i, carry):
            r0 = sid * NSLICE + i * PC
            pltpu.sync_copy(batch_hbm.at[pl.ds(r0, PC)], bidx_v)
            pltpu.sync_copy(h_hbm.at[pl.ds(r0, PC)], rows_v)
            pltpu.sync_copy(rows_v, accP_sh.at[bidx_v], add=True)
            if do_cnt:
                pltpu.sync_copy(ones_v, accC_sh.at[bidx_v], add=True)
            return carry
        lax.fori_loop(0, P_CHUNKS, body, 0)

    @pl.when(cid == 0)
    def _():
        run(hA_hbm, True)

    @pl.when(cid == 1)
    def _():
        run(hB_hbm, False)

    plsc.subcore_barrier()

    @pl.when(sid == 0)
    def _():
        @pl.when(cid == 0)
        def _():
            pltpu.sync_copy(accP_sh.at[pl.ds(0, G)], sumsA_hbm)
            pltpu.sync_copy(accC_sh.at[pl.ds(0, G)], cnt_hbm)

        @pl.when(cid == 1)
        def _():
            pltpu.sync_copy(accP_sh.at[pl.ds(0, G)], sumsB_hbm)


RB = 1024
N_BLOCKS = NPAD // RB
PK = RB // 4
NP4 = NPAD // 4


def _leaky_tc(v):
    return jnp.where(v >= 0, v, 0.01 * v)


def _t0_body(xT_ref, W_ref, xwA_ref, xwB_ref):
    xw = lax.dot_general(xT_ref[...], W_ref[...], (((0,), (0,)), ((), ())),
                         preferred_element_type=F32)
    xwA_ref[...] = xw[:, :HH]
    xwB_ref[...] = xw[:, HH:]


_t0_call = pl.pallas_call(
    _t0_body,
    grid=(N_BLOCKS,),
    in_specs=[
        pl.BlockSpec((NODE_IN, RB), lambda i: (0, i)),
        pl.BlockSpec((NODE_IN, H), lambda i: (0, 0)),
    ],
    out_specs=[
        pl.BlockSpec((RB, HH), lambda i: (i, 0)),
        pl.BlockSpec((RB, HH), lambda i: (i, 0)),
    ],
    out_shape=[
        jax.ShapeDtypeStruct((NPAD, HH), F32),
        jax.ShapeDtypeStruct((NPAD, HH), F32),
    ],
)


def _t0b_body(xwA_ref, xwB_ref, dA4_ref, R_ref,
              xsA_ref, xsB_ref, dv_ref):
    degrep = jnp.dot(dA4_ref[...], R_ref[...],
                     preferred_element_type=F32)
    dv4 = lax.rsqrt(degrep + 1.0)
    xsA_ref[...] = xwA_ref[...] * dv4
    xsB_ref[...] = xwB_ref[...] * dv4
    dv_ref[...] = dv4


_t0b_call = pl.pallas_call(
    _t0b_body,
    grid=(N_BLOCKS,),
    in_specs=[pl.BlockSpec((PK, 128), lambda i: (i, 0))] * 2 + [
        pl.BlockSpec((PK, 4), lambda i: (i, 0)),
        pl.BlockSpec((4, 128), lambda i: (0, 0)),
    ],
    out_specs=[pl.BlockSpec((PK, 128), lambda i: (i, 0))] * 3,
    out_shape=[jax.ShapeDtypeStruct((NP4, 128), F32)] * 3,
)


def _t12_body(SA_ref, SB_ref, xA_ref, xB_ref, dv_ref,
              bA_ref, bB_ref, WAA_ref, WBA_ref, WAB_ref, WBB_ref,
              outA_ref, outB_ref):
    dv4 = dv_ref[...]
    hA = _leaky_tc(dv4 * (SA_ref[...] + xA_ref[...]) + bA_ref[...])
    hB = _leaky_tc(dv4 * (SB_ref[...] + xB_ref[...]) + bB_ref[...])
    xwA = (jnp.dot(hA, WAA_ref[...], preferred_element_type=F32)
           + jnp.dot(hB, WBA_ref[...], preferred_element_type=F32))
    xwB = (jnp.dot(hA, WAB_ref[...], preferred_element_type=F32)
           + jnp.dot(hB, WBB_ref[...], preferred_element_type=F32))
    outA_ref[...] = xwA * dv4
    outB_ref[...] = xwB * dv4


_t12_call = pl.pallas_call(
    _t12_body,
    grid=(N_BLOCKS,),
    in_specs=[pl.BlockSpec((PK, 128), lambda i: (i, 0))] * 5 + [
        pl.BlockSpec((1, 128), lambda i: (0, 0)),
        pl.BlockSpec((1, 128), lambda i: (0, 0)),
        pl.BlockSpec((128, 128), lambda i: (0, 0)),
        pl.BlockSpec((128, 128), lambda i: (0, 0)),
        pl.BlockSpec((128, 128), lambda i: (0, 0)),
        pl.BlockSpec((128, 128), lambda i: (0, 0)),
    ],
    out_specs=[pl.BlockSpec((PK, 128), lambda i: (i, 0))] * 2,
    out_shape=[jax.ShapeDtypeStruct((NP4, 128), F32)] * 2,
)


def _t3_body(SA_ref, SB_ref, xA_ref, xB_ref, dv_ref, bA_ref, bB_ref,
             hA_ref, hB_ref):
    dv4 = dv_ref[...]
    hA_ref[...] = _leaky_tc(dv4 * (SA_ref[...] + xA_ref[...]) + bA_ref[...])
    hB_ref[...] = _leaky_tc(dv4 * (SB_ref[...] + xB_ref[...]) + bB_ref[...])


_t3_call = pl.pallas_call(
    _t3_body,
    grid=(N_BLOCKS,),
    in_specs=[pl.BlockSpec((PK, 128), lambda i: (i, 0))] * 5 + [
        pl.BlockSpec((1, 128), lambda i: (0, 0)),
        pl.BlockSpec((1, 128), lambda i: (0, 0)),
    ],
    out_specs=[pl.BlockSpec((PK, 128), lambda i: (i, 0))] * 2,
    out_shape=[jax.ShapeDtypeStruct((NP4, 128), F32)] * 2,
)


def _head_body(sA_ref, sB_ref, cnt_ref,
               FAA_ref, FBA_ref, FAB_ref, FBB_ref, bfA_ref, bfB_ref,
               W2A_ref, W2B_ref, bf2_ref, out_ref):
    c = jnp.maximum(cnt_ref[...], 1.0)
    pA = sA_ref[...] / c
    pB = sB_ref[...] / c
    zA = _leaky_tc(jnp.dot(pA, FAA_ref[...], preferred_element_type=F32)
                   + jnp.dot(pB, FBA_ref[...], preferred_element_type=F32)
                   + bfA_ref[...])
    zB = _leaky_tc(jnp.dot(pA, FAB_ref[...], preferred_element_type=F32)
                   + jnp.dot(pB, FBB_ref[...], preferred_element_type=F32)
                   + bfB_ref[...])
    out_ref[...] = (jnp.dot(zA, W2A_ref[...], preferred_element_type=F32)
                    + jnp.dot(zB, W2B_ref[...], preferred_element_type=F32)
                    + bf2_ref[...])


_head_call = pl.pallas_call(
    _head_body,
    out_shape=jax.ShapeDtypeStruct((G // 4, 4), F32),
)


def kernel(x, edge_index, batch, W0, b0, W1, b1, W2, b2, Wf1, bf1, Wf2, bf2):
    xT = x.T
    batch_pad = jnp.concatenate(
        [batch, jnp.full((NPAD - N,), G, dtype=I32)])
    zeros2 = jnp.zeros((NSLICE, HH), dtype=F32)
    zeros1 = jnp.zeros((NSLICE,), dtype=F32)
    ones_d = jnp.ones((DEG_EC,), dtype=F32)
    ones_p = jnp.ones((PC, HH), dtype=F32)

    def pack(a):
        return a.reshape(NP4, 128)

    def unpack(a):
        return a.reshape(NPAD, HH)

    eye4 = jnp.eye(4, dtype=F32)

    def bd(M):
        return jnp.kron(eye4, M)

    def tile4(v):
        return jnp.tile(v, 4).reshape(1, 128)

    deg = _deg_kernel(edge_index, zeros1, ones_d)
    xwA, xwB = _t0_call(xT, W0)
    Rrep = jnp.kron(eye4, jnp.ones((1, HH), dtype=F32))
    xsA_p, xsB_p, dv_p = _t0b_call(pack(xwA), pack(xwB),
                                   deg.reshape(NP4, 4), Rrep)
    SA, SB = _edge_kernel(edge_index, unpack(xsA_p), unpack(xsB_p), zeros2)
    xsA_p, xsB_p = _t12_call(
        pack(SA), pack(SB), xsA_p, xsB_p, dv_p,
        tile4(b0[:HH]), tile4(b0[HH:]),
        bd(W1[:HH, :HH]), bd(W1[HH:, :HH]), bd(W1[:HH, HH:]), bd(W1[HH:, HH:]))
    SA, SB = _edge_kernel(edge_index, unpack(xsA_p), unpack(xsB_p), zeros2)
    xsA_p, xsB_p = _t12_call(
        pack(SA), pack(SB), xsA_p, xsB_p, dv_p,
        tile4(b1[:HH]), tile4(b1[HH:]),
        bd(W2[:HH, :HH]), bd(W2[HH:, :HH]), bd(W2[:HH, HH:]), bd(W2[HH:, HH:]))
    SA, SB = _edge_kernel(edge_index, unpack(xsA_p), unpack(xsB_p), zeros2)
    hA_p, hB_p = _t3_call(pack(SA), pack(SB), xsA_p, xsB_p, dv_p,
                          tile4(b2[:HH]), tile4(b2[HH:]))
    sumsA, sumsB, cntf = _pool_kernel(unpack(hA_p), unpack(hB_p), batch_pad,
                                      zeros2, ones_p)
    out4 = _head_call(
        sumsA.reshape(G // 4, 128), sumsB.reshape(G // 4, 128),
        cntf.reshape(G // 4, 128),
        bd(Wf1[:HH, :HH]), bd(Wf1[HH:, :HH]), bd(Wf1[:HH, HH:]),
        bd(Wf1[HH:, HH:]),
        tile4(bf1[:HH]), tile4(bf1[HH:]),
        bd(Wf2[:HH, :]), bd(Wf2[HH:, :]),
        bf2.reshape(1, 1))
    return out4.reshape(G)


# speedup vs baseline: 1.2050x; 1.2050x over previous
import functools

import jax
import jax.numpy as jnp
from jax import lax
from jax.experimental import pallas as pl
from jax.experimental.pallas import tpu as pltpu
from jax.experimental.pallas import tpu_sc as plsc

F32 = jnp.float32
I32 = jnp.int32

N = 50000
E = 800000
NODE_IN = 163
H = 64
HH = H // 2
G = 512

NPAD = 50176
EC = 400
E_PER_TILE = E // 16
E_CHUNKS = E_PER_TILE // EC
DEG_EC = 1000
NSLICE = NPAD // 16
PC = 784
P_CHUNKS = NSLICE // PC
GP = 520

_mesh = plsc.VectorSubcoreMesh(core_axis_name="c", subcore_axis_name="s")


@functools.partial(
    pl.kernel,
    out_type=jax.ShapeDtypeStruct((NPAD,), F32),
    mesh=_mesh,
    compiler_params=pltpu.CompilerParams(use_tc_tiling_on_sc=False),
    scratch_types=[
        pltpu.VMEM((DEG_EC,), I32),
        pltpu.VMEM((DEG_EC,), F32),
        pltpu.VMEM_SHARED((NPAD,), F32),
    ],
)
def _deg_kernel(ei_hbm, zeros1_hbm, ones_hbm, deg_hbm,
                didx_v, ones_v, acc_sh):
    cid = lax.axis_index("c")
    sid = lax.axis_index("s")

    @pl.when(cid == 0)
    def _():
        pltpu.sync_copy(zeros1_hbm, acc_sh.at[pl.ds(sid * NSLICE, NSLICE)])
        pltpu.sync_copy(ones_hbm, ones_v)

    plsc.subcore_barrier()

    @pl.when(cid == 0)
    def _():
        def body(i, carry):
            base = sid * E_PER_TILE + i * DEG_EC
            pltpu.sync_copy(ei_hbm.at[1, pl.ds(base, DEG_EC)], didx_v)
            pltpu.sync_copy(ones_v, acc_sh.at[didx_v], add=True)
            return carry

        lax.fori_loop(0, E_PER_TILE // DEG_EC, body, 0)

    plsc.subcore_barrier()

    @pl.when(cid == 0)
    def _():
        sl = pl.ds(sid * NSLICE, NSLICE)
        pltpu.sync_copy(acc_sh.at[sl], deg_hbm.at[sl])


@functools.partial(
    pl.kernel,
    out_type=(jax.ShapeDtypeStruct((NPAD, HH), F32),
              jax.ShapeDtypeStruct((NPAD, HH), F32)),
    mesh=_mesh,
    compiler_params=pltpu.CompilerParams(use_tc_tiling_on_sc=False),
    scratch_types=[
        pltpu.VMEM((EC,), I32),
        pltpu.VMEM((EC,), I32),
        pltpu.VMEM((EC,), I32),
        pltpu.VMEM((EC,), I32),
        pltpu.VMEM((EC,), I32),
        pltpu.VMEM((EC,), I32),
        pltpu.VMEM((EC,), I32),
        pltpu.VMEM((EC,), I32),
        pltpu.VMEM((EC, HH), F32),
        pltpu.VMEM((EC, HH), F32),
        pltpu.VMEM_SHARED((NPAD, HH), F32),
        pltpu.SemaphoreType.DMA,
        pltpu.SemaphoreType.DMA,
        pltpu.SemaphoreType.DMA,
        pltpu.SemaphoreType.DMA,
        pltpu.SemaphoreType.DMA,
        pltpu.SemaphoreType.DMA,
    ],
)
def _edge_kernel(ei_hbm, xsA_hbm, xsB_hbm, zeros2_hbm,
                 outA_hbm, outB_hbm,
                 sidx0, didx0, sidx1, didx1, sidx2, didx2, sidx3, didx3,
                 rows0, rows1, acc_sh,
                 isem0, isem1, isem2, isem3, gsem0, gsem1):
    cid = lax.axis_index("c")
    sid = lax.axis_index("s")
    pltpu.sync_copy(zeros2_hbm, acc_sh.at[pl.ds(sid * NSLICE, NSLICE)])
    plsc.subcore_barrier()

    sidx = (sidx0, sidx1, sidx2, sidx3)
    didx = (didx0, didx1, didx2, didx3)
    rows = (rows0, rows1)
    isem = (isem0, isem1, isem2, isem3)
    gsem = (gsem0, gsem1)

    def run(xs_hbm):
        def I(c, b):
            base = sid * E_PER_TILE + c * EC
            pltpu.async_copy(ei_hbm.at[0, pl.ds(base, EC)], sidx[b], isem[b])
            pltpu.async_copy(ei_hbm.at[1, pl.ds(base, EC)], didx[b], isem[b])

        def Iw(c, b):
            @pl.when(c < E_CHUNKS)
            def _():
                I(c, b)

        def Gstart(c, b, r):
            base = sid * E_PER_TILE + c * EC
            pltpu.make_async_copy(ei_hbm.at[0, pl.ds(base, EC)], sidx[b],
                                  isem[b]).wait()
            pltpu.make_async_copy(ei_hbm.at[1, pl.ds(base, EC)], didx[b],
                                  isem[b]).wait()
            pltpu.async_copy(xs_hbm.at[sidx[b]], rows[r], gsem[r])

        def S(c, b, r):
            pltpu.make_async_copy(xs_hbm.at[sidx[b]], rows[r],
                                  gsem[r]).wait()
            pltpu.sync_copy(rows[r], acc_sh.at[didx[b]], add=True)

        I(0, 0)
        I(1, 1)
        I(2, 2)
        Gstart(0, 0, 0)

        def body(p, carry):
            q = 4 * p
            I(q + 3, 3)
            Gstart(q + 1, 1, 1)
            S(q, 0, 0)
            I(q + 4, 0)
            Gstart(q + 2, 2, 0)
            S(q + 1, 1, 1)
            Iw(q + 5, 1)
            Gstart(q + 3, 3, 1)
            S(q + 2, 2, 0)
            Iw(q + 6, 2)
            Gstart(q + 4, 0, 0)
            S(q + 3, 3, 1)
            Iw(q + 7, 3)
            return carry

        lax.fori_loop(0, (E_CHUNKS - 1) // 4, body, 0)
        S(E_CHUNKS - 1, 0, 0)

    @pl.when(cid == 0)
    def _():
        run(xsA_hbm)

    @pl.when(cid == 1)
    def _():
        run(xsB_hbm)

    plsc.subcore_barrier()
    sl = pl.ds(sid * NSLICE, NSLICE)

    @pl.when(cid == 0)
    def _():
        pltpu.sync_copy(acc_sh.at[sl], outA_hbm.at[sl])

    @pl.when(cid == 1)
    def _():
        pltpu.sync_copy(acc_sh.at[sl], outB_hbm.at[sl])


@functools.partial(
    pl.kernel,
    out_type=(jax.ShapeDtypeStruct((G, HH), F32),
              jax.ShapeDtypeStruct((G, HH), F32),
              jax.ShapeDtypeStruct((G, HH), F32)),
    mesh=_mesh,
    compiler_params=pltpu.CompilerParams(use_tc_tiling_on_sc=False),
    scratch_types=[
        pltpu.VMEM((PC,), I32),
        pltpu.VMEM((PC, HH), F32),
        pltpu.VMEM((PC, HH), F32),
        pltpu.VMEM_SHARED((GP, HH), F32),
        pltpu.VMEM_SHARED((GP, HH), F32),
    ],
)
def _pool_kernel(hA_hbm, hB_hbm, batch_hbm, zeros2_hbm, ones_hbm,
                 sumsA_hbm, sumsB_hbm, cnt_hbm,
                 bidx_v, rows_v, ones_v, accP_sh, accC_sh):
    cid = lax.axis_index("c")
    sid = lax.axis_index("s")

    @pl.when(sid == 0)
    def _():
        pltpu.sync_copy(zeros2_hbm.at[pl.ds(0, GP)], accP_sh)
        pltpu.sync_copy(zeros2_hbm.at[pl.ds(0, GP)], accC_sh)

    pltpu.sync_copy(ones_hbm, ones_v)
    plsc.subcore_barrier()

    def run(h_hbm, do_cnt):
        def body(i, carry):
            r0 = sid * NSLICE + i * PC
            pltpu.sync_copy(batch_hbm.at[pl.ds(r0, PC)], bidx_v)
            pltpu.sync_copy(h_hbm.at[pl.ds(r0, PC)], rows_v)
            pltpu.sync_copy(rows_v, accP_sh.at[bidx_v], add=True)
            if do_cnt:
                pltpu.sync_copy(ones_v, accC_sh.at[bidx_v], add=True)
            return carry
        lax.fori_loop(0, P_CHUNKS, body, 0)

    @pl.when(cid == 0)
    def _():
        run(hA_hbm, True)

    @pl.when(cid == 1)
    def _():
        run(hB_hbm, False)

    plsc.subcore_barrier()

    @pl.when(sid == 0)
    def _():
        @pl.when(cid == 0)
        def _():
            pltpu.sync_copy(accP_sh.at[pl.ds(0, G)], sumsA_hbm)
            pltpu.sync_copy(accC_sh.at[pl.ds(0, G)], cnt_hbm)

        @pl.when(cid == 1)
        def _():
            pltpu.sync_copy(accP_sh.at[pl.ds(0, G)], sumsB_hbm)


RB = 1024
N_BLOCKS = NPAD // RB
PK = RB // 4
NP4 = NPAD // 4


def _leaky_tc(v):
    return jnp.where(v >= 0, v, 0.01 * v)


def _t0_body(xT_ref, W_ref, xwA_ref, xwB_ref):
    xw = lax.dot_general(xT_ref[...], W_ref[...], (((0,), (0,)), ((), ())),
                         preferred_element_type=F32)
    xwA_ref[...] = xw[:, :HH]
    xwB_ref[...] = xw[:, HH:]


_t0_call = pl.pallas_call(
    _t0_body,
    grid=(N_BLOCKS,),
    in_specs=[
        pl.BlockSpec((NODE_IN, RB), lambda i: (0, i)),
        pl.BlockSpec((NODE_IN, H), lambda i: (0, 0)),
    ],
    out_specs=[
        pl.BlockSpec((RB, HH), lambda i: (i, 0)),
        pl.BlockSpec((RB, HH), lambda i: (i, 0)),
    ],
    out_shape=[
        jax.ShapeDtypeStruct((NPAD, HH), F32),
        jax.ShapeDtypeStruct((NPAD, HH), F32),
    ],
)


def _t0b_body(xwA_ref, xwB_ref, dA4_ref, R_ref,
              xsA_ref, xsB_ref, dv_ref):
    degrep = jnp.dot(dA4_ref[...], R_ref[...],
                     preferred_element_type=F32)
    dv4 = lax.rsqrt(degrep + 1.0)
    xsA_ref[...] = xwA_ref[...] * dv4
    xsB_ref[...] = xwB_ref[...] * dv4
    dv_ref[...] = dv4


_t0b_call = pl.pallas_call(
    _t0b_body,
    grid=(N_BLOCKS,),
    in_specs=[pl.BlockSpec((PK, 128), lambda i: (i, 0))] * 2 + [
        pl.BlockSpec((PK, 4), lambda i: (i, 0)),
        pl.BlockSpec((4, 128), lambda i: (0, 0)),
    ],
    out_specs=[pl.BlockSpec((PK, 128), lambda i: (i, 0))] * 3,
    out_shape=[jax.ShapeDtypeStruct((NP4, 128), F32)] * 3,
)


def _t12_body(SA_ref, SB_ref, xA_ref, xB_ref, dv_ref,
              bA_ref, bB_ref, WAA_ref, WBA_ref, WAB_ref, WBB_ref,
              outA_ref, outB_ref):
    dv4 = dv_ref[...]
    hA = _leaky_tc(dv4 * (SA_ref[...] + xA_ref[...]) + bA_ref[...])
    hB = _leaky_tc(dv4 * (SB_ref[...] + xB_ref[...]) + bB_ref[...])
    xwA = (jnp.dot(hA, WAA_ref[...], preferred_element_type=F32)
           + jnp.dot(hB, WBA_ref[...], preferred_element_type=F32))
    xwB = (jnp.dot(hA, WAB_ref[...], preferred_element_type=F32)
           + jnp.dot(hB, WBB_ref[...], preferred_element_type=F32))
    outA_ref[...] = xwA * dv4
    outB_ref[...] = xwB * dv4


_t12_call = pl.pallas_call(
    _t12_body,
    grid=(N_BLOCKS,),
    in_specs=[pl.BlockSpec((PK, 128), lambda i: (i, 0))] * 5 + [
        pl.BlockSpec((1, 128), lambda i: (0, 0)),
        pl.BlockSpec((1, 128), lambda i: (0, 0)),
        pl.BlockSpec((128, 128), lambda i: (0, 0)),
        pl.BlockSpec((128, 128), lambda i: (0, 0)),
        pl.BlockSpec((128, 128), lambda i: (0, 0)),
        pl.BlockSpec((128, 128), lambda i: (0, 0)),
    ],
    out_specs=[pl.BlockSpec((PK, 128), lambda i: (i, 0))] * 2,
    out_shape=[jax.ShapeDtypeStruct((NP4, 128), F32)] * 2,
)


def _t3_body(SA_ref, SB_ref, xA_ref, xB_ref, dv_ref, bA_ref, bB_ref,
             hA_ref, hB_ref):
    dv4 = dv_ref[...]
    hA_ref[...] = _leaky_tc(dv4 * (SA_ref[...] + xA_ref[...]) + bA_ref[...])
    hB_ref[...] = _leaky_tc(dv4 * (SB_ref[...] + xB_ref[...]) + bB_ref[...])


_t3_call = pl.pallas_call(
    _t3_body,
    grid=(N_BLOCKS,),
    in_specs=[pl.BlockSpec((PK, 128), lambda i: (i, 0))] * 5 + [
        pl.BlockSpec((1, 128), lambda i: (0, 0)),
        pl.BlockSpec((1, 128), lambda i: (0, 0)),
    ],
    out_specs=[pl.BlockSpec((PK, 128), lambda i: (i, 0))] * 2,
    out_shape=[jax.ShapeDtypeStruct((NP4, 128), F32)] * 2,
)


def _head_body(sA_ref, sB_ref, cnt_ref,
               FAA_ref, FBA_ref, FAB_ref, FBB_ref, bfA_ref, bfB_ref,
               W2A_ref, W2B_ref, bf2_ref, out_ref):
    c = jnp.maximum(cnt_ref[...], 1.0)
    pA = sA_ref[...] / c
    pB = sB_ref[...] / c
    zA = _leaky_tc(jnp.dot(pA, FAA_ref[...], preferred_element_type=F32)
                   + jnp.dot(pB, FBA_ref[...], preferred_element_type=F32)
                   + bfA_ref[...])
    zB = _leaky_tc(jnp.dot(pA, FAB_ref[...], preferred_element_type=F32)
                   + jnp.dot(pB, FBB_ref[...], preferred_element_type=F32)
                   + bfB_ref[...])
    out_ref[...] = (jnp.dot(zA, W2A_ref[...], preferred_element_type=F32)
                    + jnp.dot(zB, W2B_ref[...], preferred_element_type=F32)
                    + bf2_ref[...])


_head_call = pl.pallas_call(
    _head_body,
    out_shape=jax.ShapeDtypeStruct((G // 4, 4), F32),
)


def kernel(x, edge_index, batch, W0, b0, W1, b1, W2, b2, Wf1, bf1, Wf2, bf2):
    xT = x.T
    batch_pad = jnp.concatenate(
        [batch, jnp.full((NPAD - N,), G, dtype=I32)])
    zeros2 = jnp.zeros((NSLICE, HH), dtype=F32)
    zeros1 = jnp.zeros((NSLICE,), dtype=F32)
    ones_d = jnp.ones((DEG_EC,), dtype=F32)
    ones_p = jnp.ones((PC, HH), dtype=F32)

    def pack(a):
        return a.reshape(NP4, 128)

    def unpack(a):
        return a.reshape(NPAD, HH)

    eye4 = jnp.eye(4, dtype=F32)

    def bd(M):
        return jnp.kron(eye4, M)

    def tile4(v):
        return jnp.tile(v, 4).reshape(1, 128)

    deg = _deg_kernel(edge_index, zeros1, ones_d)
    xwA, xwB = _t0_call(xT, W0)
    Rrep = jnp.kron(eye4, jnp.ones((1, HH), dtype=F32))
    xsA_p, xsB_p, dv_p = _t0b_call(pack(xwA), pack(xwB),
                                   deg.reshape(NP4, 4), Rrep)
    SA, SB = _edge_kernel(edge_index, unpack(xsA_p), unpack(xsB_p), zeros2)
    xsA_p, xsB_p = _t12_call(
        pack(SA), pack(SB), xsA_p, xsB_p, dv_p,
        tile4(b0[:HH]), tile4(b0[HH:]),
        bd(W1[:HH, :HH]), bd(W1[HH:, :HH]), bd(W1[:HH, HH:]), bd(W1[HH:, HH:]))
    SA, SB = _edge_kernel(edge_index, unpack(xsA_p), unpack(xsB_p), zeros2)
    xsA_p, xsB_p = _t12_call(
        pack(SA), pack(SB), xsA_p, xsB_p, dv_p,
        tile4(b1[:HH]), tile4(b1[HH:]),
        bd(W2[:HH, :HH]), bd(W2[HH:, :HH]), bd(W2[:HH, HH:]), bd(W2[HH:, HH:]))
    SA, SB = _edge_kernel(edge_index, unpack(xsA_p), unpack(xsB_p), zeros2)
    hA_p, hB_p = _t3_call(pack(SA), pack(SB), xsA_p, xsB_p, dv_p,
                          tile4(b2[:HH]), tile4(b2[HH:]))
    sumsA, sumsB, cntf = _pool_kernel(unpack(hA_p), unpack(hB_p), batch_pad,
                                      zeros2, ones_p)
    out4 = _head_call(
        sumsA.reshape(G // 4, 128), sumsB.reshape(G // 4, 128),
        cntf.reshape(G // 4, 128),
        bd(Wf1[:HH, :HH]), bd(Wf1[HH:, :HH]), bd(Wf1[:HH, HH:]),
        bd(Wf1[HH:, HH:]),
        tile4(bf1[:HH]), tile4(bf1[HH:]),
        bd(Wf2[:HH, :]), bd(Wf2[HH:, :]),
        bf2.reshape(1, 1))
    return out4.reshape(G)
